# baseline (speedup 1.0000x reference)
import functools
import math

import jax
import jax.numpy as jnp
import numpy as np
from jax import lax
from jax.experimental import pallas as pl
from jax.experimental.pallas import tpu as pltpu
from jax.experimental.pallas import tpu_sc as plsc

DIM = 128
MAXP = 10000
SC_ROWS = 8192
TC_BLOCK = 512


@functools.lru_cache(maxsize=None)
def _make_sc_gather(V, D, B):
    info = plsc.get_sparse_core_info()
    NC, NS = info.num_cores, info.num_subcores
    NW = NC * NS
    assert B % (8 * NW) == 0
    b_per_w = B // NW
    mesh = plsc.VectorSubcoreMesh(core_axis_name="c", subcore_axis_name="s")

    @functools.partial(
        pl.kernel,
        mesh=mesh,
        out_type=jax.ShapeDtypeStruct((B, D), jnp.float32),
        scratch_types=[
            pltpu.VMEM((b_per_w,), jnp.int32),
            pltpu.VMEM((b_per_w, D), jnp.float32),
            pltpu.SemaphoreType.DMA,
        ],
    )
    def gather_kernel(table_hbm, idx_hbm, out_hbm, idx_v, rows_v, sem):
        wid = lax.axis_index("s") * NC + lax.axis_index("c")
        base = wid * b_per_w
        pltpu.sync_copy(idx_hbm.at[pl.ds(base, b_per_w)], idx_v)
        pltpu.async_copy(table_hbm.at[idx_v], rows_v, sem).wait()
        pltpu.sync_copy(rows_v, out_hbm.at[pl.ds(base, b_per_w)])

    return gather_kernel


def _tc_sin_body(idx_ref, freq_ref, off_ref, out_ref):
    t = idx_ref[0, 0, :].astype(jnp.float32)
    phase = t[:, None] * freq_ref[...] + off_ref[...]
    out_ref[...] = jnp.sin(phase)


@functools.lru_cache(maxsize=None)
def _make_tc_sin(V, D, B):
    assert B % TC_BLOCK == 0
    nb = B // TC_BLOCK
    return pl.pallas_call(
        _tc_sin_body,
        grid=(nb,),
        in_specs=[
            pl.BlockSpec((1, 1, TC_BLOCK), lambda i: (i, 0, 0)),
            pl.BlockSpec((1, D), lambda i: (0, 0)),
            pl.BlockSpec((1, D), lambda i: (0, 0)),
        ],
        out_specs=pl.BlockSpec((TC_BLOCK, D), lambda i: (i, 0)),
        out_shape=jax.ShapeDtypeStruct((B, D), jnp.float32),
    )


@functools.lru_cache(maxsize=None)
def _freq_off(V, D):
    half = D // 2
    freqs = np.exp(-math.log(MAXP) * np.arange(half, dtype=np.float64) / half)
    freq128 = np.repeat(freqs / (V - 1), 2).astype(np.float32)
    off128 = np.tile(np.array([0.0, math.pi / 2.0]), half).astype(np.float32)
    return jnp.asarray(freq128[None, :]), jnp.asarray(off128[None, :])


def kernel(timesteps, cache):
    V, D = cache.shape
    B = timesteps.shape[0]
    idx = timesteps.astype(jnp.int32)
    sc_out = _make_sc_gather(V, D, SC_ROWS)(cache, idx[:SC_ROWS])
    b_tc = B - SC_ROWS
    freq, off = _freq_off(V, D)
    tc_idx = idx[SC_ROWS:].reshape(b_tc // TC_BLOCK, 1, TC_BLOCK)
    tc_out = _make_tc_sin(V, D, b_tc)(tc_idx, freq, off)
    return jnp.concatenate([sc_out, tc_out], axis=0)


# speedup vs baseline: 1.1847x; 1.1847x over previous
import functools
import math

import jax
import jax.numpy as jnp
import numpy as np
from jax import lax
from jax.experimental import pallas as pl
from jax.experimental.pallas import tpu as pltpu
from jax.experimental.pallas import tpu_sc as plsc

DIM = 128
MAXP = 10000
SC_ROWS = 8192
TC_BLOCK = 512


@functools.lru_cache(maxsize=None)
def _make_sc_gather(V, D, B):
    info = plsc.get_sparse_core_info()
    NC, NS = info.num_cores, info.num_subcores
    NW = NC * NS
    assert B % (8 * NW) == 0
    b_per_w = B // NW
    mesh = plsc.VectorSubcoreMesh(core_axis_name="c", subcore_axis_name="s")

    @functools.partial(
        pl.kernel,
        mesh=mesh,
        out_type=jax.ShapeDtypeStruct((B, D), jnp.float32),
        scratch_types=[
            pltpu.VMEM((b_per_w,), jnp.int32),
            pltpu.VMEM((b_per_w, D), jnp.float32),
            pltpu.SemaphoreType.DMA,
        ],
    )
    def gather_kernel(table_hbm, idx_hbm, out_hbm, idx_v, rows_v, sem):
        wid = lax.axis_index("s") * NC + lax.axis_index("c")
        base = wid * b_per_w
        pltpu.sync_copy(idx_hbm.at[pl.ds(base, b_per_w)], idx_v)
        pltpu.async_copy(table_hbm.at[idx_v], rows_v, sem).wait()
        pltpu.sync_copy(rows_v, out_hbm.at[pl.ds(base, b_per_w)])

    return gather_kernel


def _tc_sin_body(idx_ref, freq_ref, sel_ref, out_ref):
    t = idx_ref[0, 0, :].astype(jnp.float32)
    p = t[:, None] * freq_ref[...]
    p2 = p * p
    sinp = p * (1.0 + p2 * (-1.0 / 6.0 + p2 * (1.0 / 120.0 + p2 * (-1.0 / 5040.0))))
    cosp = 1.0 + p2 * (-0.5 + p2 * (1.0 / 24.0 + p2 * (-1.0 / 720.0 + p2 / 40320.0)))
    out_ref[...] = jnp.where(sel_ref[...] > 0.0, sinp, cosp)


@functools.lru_cache(maxsize=None)
def _make_tc_sin(V, D, B):
    assert B % TC_BLOCK == 0
    nb = B // TC_BLOCK
    return pl.pallas_call(
        _tc_sin_body,
        grid=(nb,),
        in_specs=[
            pl.BlockSpec((1, 1, TC_BLOCK), lambda i: (i, 0, 0)),
            pl.BlockSpec((1, D), lambda i: (0, 0)),
            pl.BlockSpec((1, D), lambda i: (0, 0)),
        ],
        out_specs=pl.BlockSpec((TC_BLOCK, D), lambda i: (i, 0)),
        out_shape=jax.ShapeDtypeStruct((B, D), jnp.float32),
    )


@functools.lru_cache(maxsize=None)
def _freq_off(V, D):
    half = D // 2
    freqs = np.exp(-math.log(MAXP) * np.arange(half, dtype=np.float64) / half)
    freq128 = np.repeat(freqs / (V - 1), 2).astype(np.float32)
    sel128 = np.tile(np.array([1.0, -1.0]), half).astype(np.float32)
    return jnp.asarray(freq128[None, :]), jnp.asarray(sel128[None, :])


def kernel(timesteps, cache):
    V, D = cache.shape
    B = timesteps.shape[0]
    idx = timesteps.astype(jnp.int32)
    sc_out = _make_sc_gather(V, D, SC_ROWS)(cache, idx[:SC_ROWS])
    b_tc = B - SC_ROWS
    freq, off = _freq_off(V, D)
    tc_idx = idx[SC_ROWS:].reshape(b_tc // TC_BLOCK, 1, TC_BLOCK)
    tc_out = _make_tc_sin(V, D, b_tc)(tc_idx, freq, off)
    return jnp.concatenate([sc_out, tc_out], axis=0)
